# baseline (speedup 1.0000x reference)
import dataclasses

import jax
import jax.numpy as jnp
from jax import lax
from jax.experimental import pallas as pl
from jax.experimental.pallas import tpu as pltpu
from jax.experimental.pallas import tpu_sc as plsc

_BM = 2048
_K = 8
_BQ = _BM // _K

_B_SC = 4096
_NW = 32
_RPW = _B_SC // _NW
_NPAIRS = _RPW // 2
_L = 16


def _tc_block(t_ref, *refs):
    x_refs = refs[:_K]
    lbl_ref, w_ref, out_ref = refs[_K], refs[_K + 1], refs[_K + 2]
    inv_t = 1.0 / t_ref[0]
    part = jnp.float32(0.0)
    for k in range(_K):
        x = x_refs[k][...]
        e = jnp.exp(x * inv_t)
        s1 = jnp.sum(e, axis=1, keepdims=True)
        s2 = jnp.sum(e * e, axis=1, keepdims=True)
        cols = jax.lax.broadcasted_iota(jnp.int32, x.shape, 1)
        lbl = lbl_ref[k * _BQ:(k + 1) * _BQ, :]
        el = jnp.sum(jnp.where(cols == lbl, e, 0.0), axis=1, keepdims=True)
        brier = (s2 + (s1 - 2.0 * el) * s1) / (s1 * s1)
        part = part + jnp.sum(brier * w_ref[k * _BQ:(k + 1) * _BQ, :])
    prev = jnp.where(pl.program_id(0) == 0, 0.0, out_ref[0, 0])
    out_ref[...] = jnp.full((8, 128), prev + part, jnp.float32)


def _tc_partial(logits, lbl2d, weight, T, b_tc, C):
    grid = b_tc // _BM
    x_specs = [
        pl.BlockSpec((_BQ, C), lambda i, k=k: (i * _K + k, 0)) for k in range(_K)
    ]
    acc = pl.pallas_call(
        _tc_block,
        grid=(grid,),
        in_specs=[pl.BlockSpec(memory_space=pltpu.SMEM)]
        + x_specs
        + [
            pl.BlockSpec((_BM, 1), lambda i: (i, 0)),
            pl.BlockSpec((_BM, 1), lambda i: (i, 0)),
        ],
        out_specs=pl.BlockSpec((8, 128), lambda i: (0, 0)),
        out_shape=jax.ShapeDtypeStruct((8, 128), jnp.float32),
    )(T, *([logits] * _K), lbl2d, weight)
    return acc[0, 0]


def _sc_partial(flat_logits, labf, wf, t16, b_tc, C):
    mesh = plsc.VectorSubcoreMesh(core_axis_name="c", subcore_axis_name="s")
    pair_elems = 2 * C
    cparams = pltpu.CompilerParams()
    if "needs_layout_passes" in pltpu.CompilerParams.__dataclass_fields__:
        cparams = dataclasses.replace(cparams, needs_layout_passes=False)

    @pl.kernel(
        out_type=jax.ShapeDtypeStruct((_NW, _L), jnp.float32),
        mesh=mesh,
        compiler_params=cparams,
        scratch_types=[
            pltpu.VMEM((pair_elems,), jnp.float32),
            pltpu.VMEM((pair_elems,), jnp.float32),
            pltpu.VMEM((2 * _L,), jnp.int32),
            pltpu.VMEM((2 * _L,), jnp.int32),
            pltpu.VMEM((2 * _L,), jnp.float32),
            pltpu.VMEM((2 * _L,), jnp.float32),
            pltpu.VMEM((_L,), jnp.float32),
            pltpu.VMEM((_L,), jnp.float32),
            pltpu.SemaphoreType.DMA,
            pltpu.SemaphoreType.DMA,
            pltpu.SemaphoreType.DMA,
            pltpu.SemaphoreType.DMA,
            pltpu.SemaphoreType.DMA,
            pltpu.SemaphoreType.DMA,
        ],
    )
    def sc_kernel(x_hbm, lbl_hbm, w_hbm, t_hbm, out_hbm,
                  buf_a, buf_b, lab_a, lab_b, wv_a, wv_b, t_v, tot_v,
                  sx_a, sx_b, sl_a, sl_b, sw_a, sw_b):
        iota = lax.iota(jnp.int32, _L)
        wid = lax.axis_index("s") * 2 + lax.axis_index("c")
        lrow0 = wid * _RPW
        elem0 = (b_tc + lrow0) * C

        pltpu.sync_copy(t_hbm, t_v)
        invt = 1.0 / t_v[pl.ds(0, _L)]
        tot_v[pl.ds(0, _L)] = jnp.zeros((_L,), jnp.float32)

        def sweep(buf, lvA, lvB):
            zero = jnp.zeros((_L,), jnp.float32)
            s1a = s2a = ela = s1b = s2b = elb = zero
            for j in range(pair_elems // _L):
                v = buf[pl.ds(j * _L, _L)]
                e = jnp.exp(v * invt)
                e2 = e * e
                lo = j * _L
                if lo + _L <= C:
                    s1a = s1a + e
                    s2a = s2a + e2
                    ela = ela + jnp.where((iota + lo) == lvA, e, 0.0)
                elif lo >= C:
                    s1b = s1b + e
                    s2b = s2b + e2
                    elb = elb + jnp.where((iota + (lo - C)) == lvB, e, 0.0)
                else:
                    in_a = iota < (C - lo)
                    s1a = s1a + jnp.where(in_a, e, 0.0)
                    s2a = s2a + jnp.where(in_a, e2, 0.0)
                    s1b = s1b + jnp.where(in_a, 0.0, e)
                    s2b = s2b + jnp.where(in_a, 0.0, e2)
                    ela = ela + jnp.where((iota + lo) == lvA, e, 0.0)
                    elb = elb + jnp.where((iota + (lo - C)) == lvB, e, 0.0)
            return (jnp.sum(s1a, axis=0), jnp.sum(s2a, axis=0),
                    jnp.sum(ela, axis=0), jnp.sum(s1b, axis=0),
                    jnp.sum(s2b, axis=0), jnp.sum(elb, axis=0))

        def brier(s1, s2, el):
            s1v = jnp.full((_L,), s1)
            bv = (jnp.full((_L,), s2) + (s1v - 2.0 * jnp.full((_L,), el)) * s1v) / (s1v * s1v)
            return jnp.max(bv, axis=0)

        def accum(buf, lab, wv):
            lvA = lab[pl.ds(0, _L)]
            lvB = lab[pl.ds(_L, _L)]
            s1a, s2a, ela, s1b, s2b, elb = sweep(buf, lvA, lvB)
            br_a = brier(s1a, s2a, ela)
            br_b = brier(s1b, s2b, elb)
            wa = jnp.max(wv[pl.ds(0, _L)], axis=0)
            wb = jnp.max(wv[pl.ds(_L, _L)], axis=0)
            tot = tot_v[pl.ds(0, _L)]
            tot_v[pl.ds(0, _L)] = tot + jnp.where(
                iota == 0, br_a * wa + br_b * wb, 0.0)

        def start(p, buf, lab, wv, sx, sl, sw):
            pltpu.async_copy(
                x_hbm.at[pl.ds(elem0 + p * pair_elems, pair_elems)], buf, sx)
            pltpu.async_copy(
                lbl_hbm.at[pl.ds((lrow0 + 2 * p) * _L, 2 * _L)], lab, sl)
            pltpu.async_copy(
                w_hbm.at[pl.ds((lrow0 + 2 * p) * _L, 2 * _L)], wv, sw)

        def wait(buf, lab, wv, sx, sl, sw):
            pltpu.make_async_copy(
                x_hbm.at[pl.ds(elem0, pair_elems)], buf, sx).wait()
            pltpu.make_async_copy(
                lbl_hbm.at[pl.ds(lrow0 * _L, 2 * _L)], lab, sl).wait()
            pltpu.make_async_copy(
                w_hbm.at[pl.ds(lrow0 * _L, 2 * _L)], wv, sw).wait()

        start(0, buf_a, lab_a, wv_a, sx_a, sl_a, sw_a)
        start(1, buf_b, lab_b, wv_b, sx_b, sl_b, sw_b)

        @pl.loop(0, _NPAIRS, step=2)
        def _(p2):
            wait(buf_a, lab_a, wv_a, sx_a, sl_a, sw_a)
            accum(buf_a, lab_a, wv_a)

            @pl.when(p2 + 2 < _NPAIRS)
            def _():
                start(p2 + 2, buf_a, lab_a, wv_a, sx_a, sl_a, sw_a)

            wait(buf_b, lab_b, wv_b, sx_b, sl_b, sw_b)
            accum(buf_b, lab_b, wv_b)

            @pl.when(p2 + 3 < _NPAIRS)
            def _():
                start(p2 + 3, buf_b, lab_b, wv_b, sx_b, sl_b, sw_b)

        pltpu.sync_copy(tot_v, out_hbm.at[wid])

    out = sc_kernel(flat_logits, labf, wf, t16)
    return jnp.sum(out[:, 0])


def kernel(logits, labels, weight, T):
    B, C = logits.shape
    b_tc = B - _B_SC
    lbl = labels.astype(jnp.int32)
    tc_part = _tc_partial(logits, lbl.reshape(B, 1), weight, T, b_tc, C)
    labf = jnp.broadcast_to(lbl[b_tc:, None], (_B_SC, _L)).reshape(-1)
    wf = jnp.broadcast_to(weight[b_tc:], (_B_SC, _L)).reshape(-1)
    t16 = jnp.broadcast_to(T, (_L,))
    sc_part = _sc_partial(logits.reshape(B * C), labf, wf, t16, b_tc, C)
    return (tc_part + sc_part) / B


# speedup vs baseline: 1.0064x; 1.0064x over previous
import dataclasses

import jax
import jax.numpy as jnp
from jax import lax
from jax.experimental import pallas as pl
from jax.experimental.pallas import tpu as pltpu
from jax.experimental.pallas import tpu_sc as plsc

_BM = 2048
_K = 8
_BQ = _BM // _K

_B_SC = 4096
_NW = 32
_RPW = _B_SC // _NW
_NPAIRS = _RPW // 2
_L = 16


def _tc_block(t_ref, *refs):
    x_refs = refs[:_K]
    lbl_ref, w_ref, out_ref = refs[_K], refs[_K + 1], refs[_K + 2]
    inv_t = 1.0 / t_ref[0]
    part = jnp.float32(0.0)
    for k in range(_K):
        x = x_refs[k][...]
        e = jnp.exp(x * inv_t)
        s1 = jnp.sum(e, axis=1, keepdims=True)
        s2 = jnp.sum(e * e, axis=1, keepdims=True)
        cols = jax.lax.broadcasted_iota(jnp.int32, x.shape, 1)
        lbl = lbl_ref[k * _BQ:(k + 1) * _BQ, :]
        el = jnp.sum(jnp.where(cols == lbl, e, 0.0), axis=1, keepdims=True)
        brier = (s2 + (s1 - 2.0 * el) * s1) / (s1 * s1)
        part = part + jnp.sum(brier * w_ref[k * _BQ:(k + 1) * _BQ, :])
    prev = jnp.where(pl.program_id(0) == 0, 0.0, out_ref[0, 0])
    out_ref[...] = jnp.full((8, 128), prev + part, jnp.float32)


def _tc_partial(logits, lbl2d, weight, T, b_tc, C):
    grid = b_tc // _BM
    x_specs = [
        pl.BlockSpec((_BQ, C), lambda i, k=k: (i * _K + k, 0)) for k in range(_K)
    ]
    acc = pl.pallas_call(
        _tc_block,
        grid=(grid,),
        in_specs=[pl.BlockSpec(memory_space=pltpu.SMEM)]
        + x_specs
        + [
            pl.BlockSpec((_BM, 1), lambda i: (i, 0)),
            pl.BlockSpec((_BM, 1), lambda i: (i, 0)),
        ],
        out_specs=pl.BlockSpec((8, 128), lambda i: (0, 0)),
        out_shape=jax.ShapeDtypeStruct((8, 128), jnp.float32),
    )(T, *([logits] * _K), lbl2d, weight)
    return acc[0, 0]


def _sc_partial(flat_logits, labf, wf, t16, b_tc, C):
    mesh = plsc.VectorSubcoreMesh(core_axis_name="c", subcore_axis_name="s")
    pair_elems = 2 * C
    cparams = pltpu.CompilerParams()
    if "needs_layout_passes" in pltpu.CompilerParams.__dataclass_fields__:
        cparams = dataclasses.replace(cparams, needs_layout_passes=False)

    @pl.kernel(
        out_type=jax.ShapeDtypeStruct((_NW, _L), jnp.float32),
        mesh=mesh,
        compiler_params=cparams,
        scratch_types=[
            pltpu.VMEM((pair_elems,), jnp.float32),
            pltpu.VMEM((pair_elems,), jnp.float32),
            pltpu.VMEM((2 * _L,), jnp.int32),
            pltpu.VMEM((2 * _L,), jnp.int32),
            pltpu.VMEM((2 * _L,), jnp.float32),
            pltpu.VMEM((2 * _L,), jnp.float32),
            pltpu.VMEM((_L,), jnp.float32),
            pltpu.VMEM((_L,), jnp.float32),
            pltpu.SemaphoreType.DMA,
            pltpu.SemaphoreType.DMA,
            pltpu.SemaphoreType.DMA,
            pltpu.SemaphoreType.DMA,
            pltpu.SemaphoreType.DMA,
            pltpu.SemaphoreType.DMA,
        ],
    )
    def sc_kernel(x_hbm, lbl_hbm, w_hbm, t_hbm, out_hbm,
                  buf_a, buf_b, lab_a, lab_b, wv_a, wv_b, t_v, tot_v,
                  sx_a, sx_b, sl_a, sl_b, sw_a, sw_b):
        iota = lax.iota(jnp.int32, _L)
        wid = lax.axis_index("s") * 2 + lax.axis_index("c")
        lrow0 = wid * _RPW
        elem0 = (b_tc + lrow0) * C

        pltpu.sync_copy(t_hbm, t_v)
        invt = 1.0 / t_v[pl.ds(0, _L)]
        tot_v[pl.ds(0, _L)] = jnp.zeros((_L,), jnp.float32)

        def sweep(buf, lvA, lvB):
            zero = jnp.zeros((_L,), jnp.float32)
            s1a = s2a = ela = s1b = s2b = elb = zero
            for j in range(pair_elems // _L):
                v = buf[pl.ds(j * _L, _L)]
                e = jnp.exp(v * invt)
                e2 = e * e
                lo = j * _L
                if lo + _L <= C:
                    s1a = s1a + e
                    s2a = s2a + e2
                    ela = ela + jnp.where((iota + lo) == lvA, e, 0.0)
                elif lo >= C:
                    s1b = s1b + e
                    s2b = s2b + e2
                    elb = elb + jnp.where((iota + (lo - C)) == lvB, e, 0.0)
                else:
                    in_a = iota < (C - lo)
                    s1a = s1a + jnp.where(in_a, e, 0.0)
                    s2a = s2a + jnp.where(in_a, e2, 0.0)
                    s1b = s1b + jnp.where(in_a, 0.0, e)
                    s2b = s2b + jnp.where(in_a, 0.0, e2)
                    ela = ela + jnp.where((iota + lo) == lvA, e, 0.0)
                    elb = elb + jnp.where((iota + (lo - C)) == lvB, e, 0.0)
            return (jnp.sum(s1a, axis=0), jnp.sum(s2a, axis=0),
                    jnp.sum(ela, axis=0), jnp.sum(s1b, axis=0),
                    jnp.sum(s2b, axis=0), jnp.sum(elb, axis=0))

        def brier(s1, s2, el):
            s1v = jnp.full((_L,), s1)
            bv = (jnp.full((_L,), s2) + (s1v - 2.0 * jnp.full((_L,), el)) * s1v) / (s1v * s1v)
            return jnp.max(bv, axis=0)

        def accum(buf, lab, wv):
            lvA = lab[pl.ds(0, _L)]
            lvB = lab[pl.ds(_L, _L)]
            s1a, s2a, ela, s1b, s2b, elb = sweep(buf, lvA, lvB)
            br_a = brier(s1a, s2a, ela)
            br_b = brier(s1b, s2b, elb)
            wa = jnp.max(wv[pl.ds(0, _L)], axis=0)
            wb = jnp.max(wv[pl.ds(_L, _L)], axis=0)
            tot = tot_v[pl.ds(0, _L)]
            tot_v[pl.ds(0, _L)] = tot + jnp.where(
                iota == 0, br_a * wa + br_b * wb, 0.0)

        def start(p, buf, lab, wv, sx, sl, sw):
            pltpu.async_copy(
                x_hbm.at[pl.ds(elem0 + p * pair_elems, pair_elems)], buf, sx)
            pltpu.async_copy(
                lbl_hbm.at[pl.ds((lrow0 + 2 * p) * _L, 2 * _L)], lab, sl)
            pltpu.async_copy(
                w_hbm.at[pl.ds((lrow0 + 2 * p) * _L, 2 * _L)], wv, sw)

        def wait(buf, lab, wv, sx, sl, sw):
            pltpu.make_async_copy(
                x_hbm.at[pl.ds(elem0, pair_elems)], buf, sx).wait()
            pltpu.make_async_copy(
                lbl_hbm.at[pl.ds(lrow0 * _L, 2 * _L)], lab, sl).wait()
            pltpu.make_async_copy(
                w_hbm.at[pl.ds(lrow0 * _L, 2 * _L)], wv, sw).wait()

        start(0, buf_a, lab_a, wv_a, sx_a, sl_a, sw_a)
        start(1, buf_b, lab_b, wv_b, sx_b, sl_b, sw_b)

        @pl.loop(0, _NPAIRS, step=2)
        def _(p2):
            wait(buf_a, lab_a, wv_a, sx_a, sl_a, sw_a)
            accum(buf_a, lab_a, wv_a)

            @pl.when(p2 + 2 < _NPAIRS)
            def _():
                start(p2 + 2, buf_a, lab_a, wv_a, sx_a, sl_a, sw_a)

            wait(buf_b, lab_b, wv_b, sx_b, sl_b, sw_b)
            accum(buf_b, lab_b, wv_b)

            @pl.when(p2 + 3 < _NPAIRS)
            def _():
                start(p2 + 3, buf_b, lab_b, wv_b, sx_b, sl_b, sw_b)

        pltpu.sync_copy(tot_v, out_hbm.at[wid])

    out = sc_kernel(flat_logits, labf, wf, t16)
    return jnp.sum(out[:, 0])


def kernel(logits, labels, weight, T):
    B, C = logits.shape
    b_tc = B - _B_SC
    lbl = labels.astype(jnp.int32)
    labf = jnp.broadcast_to(lbl[b_tc:, None], (_B_SC, _L)).reshape(-1)
    wf = jnp.broadcast_to(weight[b_tc:], (_B_SC, _L)).reshape(-1)
    t16 = jnp.broadcast_to(T, (_L,))
    sc_part = _sc_partial(logits.reshape(B * C), labf, wf, t16, b_tc, C)
    tc_part = _tc_partial(logits, lbl.reshape(B, 1), weight, T, b_tc, C)
    return (tc_part + sc_part) / B


# speedup vs baseline: 1.4996x; 1.4901x over previous
import dataclasses

import jax
import jax.numpy as jnp
from jax import lax
from jax.experimental import pallas as pl
from jax.experimental.pallas import tpu as pltpu
from jax.experimental.pallas import tpu_sc as plsc

_BM = 2048
_K = 8
_BQ = _BM // _K

_B_SC = 4096
_NW = 32
_L = 16
_RG = 8


def _tc_block(t_ref, *refs):
    x_refs = refs[:_K]
    lbl_ref, w_ref, out_ref = refs[_K], refs[_K + 1], refs[_K + 2]
    inv_t = 1.0 / t_ref[0]
    part = jnp.float32(0.0)
    for k in range(_K):
        x = x_refs[k][...]
        e = jnp.exp(x * inv_t)
        s1 = jnp.sum(e, axis=1, keepdims=True)
        s2 = jnp.sum(e * e, axis=1, keepdims=True)
        cols = jax.lax.broadcasted_iota(jnp.int32, x.shape, 1)
        lbl = lbl_ref[k * _BQ:(k + 1) * _BQ, :]
        el = jnp.sum(jnp.where(cols == lbl, e, 0.0), axis=1, keepdims=True)
        brier = (s2 + (s1 - 2.0 * el) * s1) / (s1 * s1)
        part = part + jnp.sum(brier * w_ref[k * _BQ:(k + 1) * _BQ, :])
    prev = jnp.where(pl.program_id(0) == 0, 0.0, out_ref[0, 0])
    out_ref[...] = jnp.full((8, 128), prev + part, jnp.float32)


def _tc_partial(logits, lbl2d, weight, T, b_tc, C):
    grid = b_tc // _BM
    x_specs = [
        pl.BlockSpec((_BQ, C), lambda i, k=k: (i * _K + k, 0)) for k in range(_K)
    ]
    acc = pl.pallas_call(
        _tc_block,
        grid=(grid,),
        in_specs=[pl.BlockSpec(memory_space=pltpu.SMEM)]
        + x_specs
        + [
            pl.BlockSpec((_BM, 1), lambda i: (i, 0)),
            pl.BlockSpec((_BM, 1), lambda i: (i, 0)),
        ],
        out_specs=pl.BlockSpec((8, 128), lambda i: (0, 0)),
        out_shape=jax.ShapeDtypeStruct((8, 128), jnp.float32),
    )(T, *([logits] * _K), lbl2d, weight)
    return acc[0, 0]


def _sc_partial(logits, labf, wf, t16, b_tc, C):
    mesh = plsc.VectorSubcoreMesh(core_axis_name="c", subcore_axis_name="s")
    cparams = pltpu.CompilerParams()
    if "needs_layout_passes" in pltpu.CompilerParams.__dataclass_fields__:
        cparams = dataclasses.replace(cparams, needs_layout_passes=False)
    nblocks = _B_SC // _RG
    nchunks = C // _L
    g0 = b_tc // _RG

    @pl.kernel(
        out_type=jax.ShapeDtypeStruct((_NW, _L), jnp.float32),
        mesh=mesh,
        compiler_params=cparams,
        scratch_types=[
            pltpu.VMEM((_L,), jnp.float32),
            pltpu.VMEM((_L,), jnp.float32),
        ],
    )
    def sc_kernel(x_hbm, lbl_hbm, w_hbm, t_hbm, out_hbm, t_v, tot_v):
        iota = lax.iota(jnp.int32, _L)
        wid = lax.axis_index("s") * 2 + lax.axis_index("c")

        pltpu.sync_copy(t_hbm, t_v)
        invt = 1.0 / t_v[pl.ds(0, _L)]
        tot_v[pl.ds(0, _L)] = jnp.zeros((_L,), jnp.float32)

        def brier(s1, s2, el):
            s1v = jnp.full((_L,), s1)
            bv = (jnp.full((_L,), s2)
                  + (s1v - 2.0 * jnp.full((_L,), el)) * s1v) / (s1v * s1v)
            return jnp.max(bv, axis=0)

        def body(x_vmem, lab_vmem, w_vmem):
            for r in range(_RG):
                lv = lab_vmem[pl.ds(r * _L, _L)]
                zero = jnp.zeros((_L,), jnp.float32)
                s1 = s2 = el = zero
                for j in range(nchunks):
                    v = x_vmem[r, pl.ds(j * _L, _L)]
                    e = jnp.exp(v * invt)
                    s1 = s1 + e
                    s2 = s2 + e * e
                    el = el + jnp.where((iota + j * _L) == lv, e, 0.0)
                v = x_vmem[r, pl.ds(C - _L, _L)]
                e = jnp.exp(v * invt)
                valid = iota >= (nchunks * _L - (C - _L))
                e = jnp.where(valid, e, 0.0)
                s1 = s1 + e
                s2 = s2 + e * e
                el = el + jnp.where((iota + (C - _L)) == lv, e, 0.0)
                br = brier(jnp.sum(s1, axis=0), jnp.sum(s2, axis=0),
                           jnp.sum(el, axis=0))
                w = jnp.max(w_vmem[pl.ds(r * _L, _L)], axis=0)
                tot = tot_v[pl.ds(0, _L)]
                tot_v[pl.ds(0, _L)] = tot + jnp.where(iota == 0, br * w, 0.0)

        pltpu.emit_pipeline(
            body,
            grid=(nblocks,),
            in_specs=[
                pl.BlockSpec((_RG, C), index_map=lambda i: (g0 + i, 0)),
                pl.BlockSpec((_RG * _L,), index_map=lambda i: (i,)),
                pl.BlockSpec((_RG * _L,), index_map=lambda i: (i,)),
            ],
            core_axis_name=("c", "s"),
            dimension_semantics=(pltpu.PARALLEL,),
        )(x_hbm, lbl_hbm, w_hbm)

        pltpu.sync_copy(tot_v, out_hbm.at[wid])

    out = sc_kernel(logits, labf, wf, t16)
    return jnp.sum(out[:, 0])


def kernel(logits, labels, weight, T):
    B, C = logits.shape
    b_tc = B - _B_SC
    lbl = labels.astype(jnp.int32)
    labf = jnp.broadcast_to(lbl[b_tc:, None], (_B_SC, _L)).reshape(-1)
    wf = jnp.broadcast_to(weight[b_tc:], (_B_SC, _L)).reshape(-1)
    t16 = jnp.broadcast_to(T, (_L,))
    sc_part = _sc_partial(logits, labf, wf, t16, b_tc, C)
    tc_part = _tc_partial(logits, lbl.reshape(B, 1), weight, T, b_tc, C)
    return (tc_part + sc_part) / B


# speedup vs baseline: 5.9670x; 3.9791x over previous
import jax
import jax.numpy as jnp
from jax.experimental import pallas as pl
from jax.experimental.pallas import tpu as pltpu

_BN = 1024
_K = 8
_BQ = _BN // _K


def _tc_block(t_ref, *refs):
    x_refs = refs[:_K]
    lbl_ref, w_ref, out_ref = refs[_K], refs[_K + 1], refs[_K + 2]
    C = x_refs[0].shape[0]
    inv_t = 1.0 / t_ref[0]
    part = jnp.float32(0.0)
    for k in range(_K):
        x = x_refs[k][...]
        e = jnp.exp(x * inv_t)
        s1 = jnp.sum(e, axis=0, keepdims=True)
        s2 = jnp.sum(e * e, axis=0, keepdims=True)
        rows = jax.lax.broadcasted_iota(jnp.int32, x.shape, 0)
        lbl = lbl_ref[:, k * _BQ:(k + 1) * _BQ]
        el = jnp.sum(jnp.where(rows == lbl, e, 0.0), axis=0, keepdims=True)
        brier = (s2 + (s1 - 2.0 * el) * s1) / (s1 * s1)
        part = part + jnp.sum(brier * w_ref[:, k * _BQ:(k + 1) * _BQ])
    prev = jnp.where(pl.program_id(0) == 0, 0.0, out_ref[0, 0])
    out_ref[...] = jnp.full((8, 128), prev + part, jnp.float32)


def kernel(logits, labels, weight, T):
    B, C = logits.shape
    xt = logits.T
    lbl = labels.astype(jnp.int32).reshape(1, B)
    wt = weight.reshape(1, B)
    grid = B // _BN
    x_specs = [
        pl.BlockSpec((C, _BQ), lambda i, k=k: (0, i * _K + k)) for k in range(_K)
    ]
    acc = pl.pallas_call(
        _tc_block,
        grid=(grid,),
        in_specs=[pl.BlockSpec(memory_space=pltpu.SMEM)]
        + x_specs
        + [
            pl.BlockSpec((1, _BN), lambda i: (0, i)),
            pl.BlockSpec((1, _BN), lambda i: (0, i)),
        ],
        out_specs=pl.BlockSpec((8, 128), lambda i: (0, 0)),
        out_shape=jax.ShapeDtypeStruct((8, 128), jnp.float32),
    )(T, *([xt] * _K), lbl, wt)
    return acc[0, 0] / B


# speedup vs baseline: 6.5058x; 1.0903x over previous
import jax
import jax.numpy as jnp
from jax.experimental import pallas as pl
from jax.experimental.pallas import tpu as pltpu

_BN = 2048
_K = 8
_BQ = _BN // _K


def _tc_block(t_ref, *refs):
    x_refs = refs[:_K]
    lbl_ref, w_ref, out_ref = refs[_K], refs[_K + 1], refs[_K + 2]
    C = x_refs[0].shape[0]
    inv_t = 1.0 / t_ref[0]
    part = jnp.float32(0.0)
    for k in range(_K):
        x = x_refs[k][...]
        e = jnp.exp(x * inv_t)
        s1 = jnp.sum(e, axis=0, keepdims=True)
        s2 = jnp.sum(e * e, axis=0, keepdims=True)
        rows = jax.lax.broadcasted_iota(jnp.int32, x.shape, 0)
        lbl = lbl_ref[:, k * _BQ:(k + 1) * _BQ]
        el = jnp.sum(jnp.where(rows == lbl, e, 0.0), axis=0, keepdims=True)
        brier = (s2 + (s1 - 2.0 * el) * s1) / (s1 * s1)
        part = part + jnp.sum(brier * w_ref[:, k * _BQ:(k + 1) * _BQ])
    prev = jnp.where(pl.program_id(0) == 0, 0.0, out_ref[0, 0])
    out_ref[...] = jnp.full((8, 128), prev + part, jnp.float32)


def kernel(logits, labels, weight, T):
    B, C = logits.shape
    xt = logits.T
    lbl = labels.astype(jnp.int32).reshape(1, B)
    wt = weight.reshape(1, B)
    grid = B // _BN
    x_specs = [
        pl.BlockSpec((C, _BQ), lambda i, k=k: (0, i * _K + k)) for k in range(_K)
    ]
    acc = pl.pallas_call(
        _tc_block,
        grid=(grid,),
        in_specs=[pl.BlockSpec(memory_space=pltpu.SMEM)]
        + x_specs
        + [
            pl.BlockSpec((1, _BN), lambda i: (0, i)),
            pl.BlockSpec((1, _BN), lambda i: (0, i)),
        ],
        out_specs=pl.BlockSpec((8, 128), lambda i: (0, 0)),
        out_shape=jax.ShapeDtypeStruct((8, 128), jnp.float32),
    )(T, *([xt] * _K), lbl, wt)
    return acc[0, 0] / B
